# baseline (device time: 80050 ns/iter reference)
import jax
import jax.numpy as jnp
from jax import lax
from jax.experimental import pallas as pl
from jax.experimental.pallas import tpu as pltpu

N_Z = 4
N_SLOT = 2 * (N_Z - 1)


def kernel(x):
    m, n_full = x.shape
    n = n_full // N_Z
    hm = m // 2

    def body(x_ref, out_ref, send_sems, recv_sems):
        my_x = lax.axis_index("x")
        my_y = lax.axis_index("y")
        my_z = lax.axis_index("z")

        barrier_sem = pltpu.get_barrier_semaphore()
        for d in range(1, N_Z):
            q = lax.rem(my_z + d, N_Z)
            for c in range(2):
                pl.semaphore_signal(
                    barrier_sem, inc=1,
                    device_id=(c, my_y, q),
                    device_id_type=pl.DeviceIdType.MESH,
                )
        pl.semaphore_wait(barrier_sem, N_SLOT)

        rdmas = []
        for d in range(1, N_Z):
            q = lax.rem(my_z + d, N_Z)
            for c in range(2):
                rdma = pltpu.make_async_remote_copy(
                    src_ref=x_ref.at[pl.ds(my_x * hm, hm), pl.ds(q * n, n)],
                    dst_ref=out_ref.at[pl.ds(my_z * m + my_x * hm, hm), :],
                    send_sem=send_sems.at[2 * (d - 1) + c],
                    recv_sem=recv_sems.at[2 * ((N_Z - 1) - d) + my_x],
                    device_id=(c, my_y, q),
                    device_id_type=pl.DeviceIdType.MESH,
                )
                rdma.start()
                rdmas.append(rdma)

        out_ref[pl.ds(my_z * m, m), :] = x_ref[:, pl.ds(my_z * n, n)]

        for s in range(N_Z - 1):
            p = lax.rem(my_z + s + 1, N_Z)
            for c in range(2):
                recv = pltpu.make_async_remote_copy(
                    src_ref=x_ref.at[pl.ds(c * hm, hm), pl.ds(p * n, n)],
                    dst_ref=out_ref.at[pl.ds(p * m + c * hm, hm), :],
                    send_sem=send_sems.at[2 * s + c],
                    recv_sem=recv_sems.at[2 * s + c],
                    device_id=(c, my_y, p),
                    device_id_type=pl.DeviceIdType.MESH,
                )
                recv.wait_recv()

        for rdma in rdmas:
            rdma.wait_send()

    return pl.pallas_call(
        body,
        out_shape=jax.ShapeDtypeStruct((N_Z * m, n), x.dtype),
        in_specs=[pl.BlockSpec(memory_space=pltpu.VMEM)],
        out_specs=pl.BlockSpec(memory_space=pltpu.VMEM),
        scratch_shapes=[
            pltpu.SemaphoreType.DMA((N_SLOT,)),
            pltpu.SemaphoreType.DMA((N_SLOT,)),
        ],
        compiler_params=pltpu.CompilerParams(collective_id=0),
    )(x)


# device time: 60854 ns/iter; 1.3154x vs baseline; 1.3154x over previous
import jax
import jax.numpy as jnp
from jax import lax
from jax.experimental import pallas as pl
from jax.experimental.pallas import tpu as pltpu

N_Z = 4
N_REP = 4


def kernel(x):
    m, n_full = x.shape
    n = n_full // N_Z
    qm = m // N_REP

    def body(x_ref, out_ref, z_send, z_recv, x_send, x_recv,
             y_send, y_recv, g_send, g_recv):
        my_x = lax.axis_index("x")
        my_y = lax.axis_index("y")
        my_z = lax.axis_index("z")
        r_me = 2 * my_x + my_y

        barrier_sem = pltpu.get_barrier_semaphore()
        for d in range(1, N_Z):
            q = lax.rem(my_z + d, N_Z)
            pl.semaphore_signal(
                barrier_sem, inc=1,
                device_id=(my_x, my_y, q),
                device_id_type=pl.DeviceIdType.MESH,
            )
        for dev in ((1 - my_x, my_y, my_z), (my_x, 1 - my_y, my_z),
                    (1 - my_x, 1 - my_y, my_z)):
            pl.semaphore_signal(
                barrier_sem, inc=1,
                device_id=dev,
                device_id_type=pl.DeviceIdType.MESH,
            )
        pl.semaphore_wait(barrier_sem, 6)

        z_rdmas = {}
        for d in range(1, N_Z):
            q = lax.rem(my_z + d, N_Z)
            rdma = pltpu.make_async_remote_copy(
                src_ref=x_ref.at[pl.ds(r_me * qm, qm), pl.ds(q * n, n)],
                dst_ref=out_ref.at[pl.ds(my_z * m + r_me * qm, qm), :],
                send_sem=z_send.at[d - 1],
                recv_sem=z_recv.at[(N_Z - 1) - d],
                device_id=(my_x, my_y, q),
                device_id_type=pl.DeviceIdType.MESH,
            )
            rdma.start()
            z_rdmas[d] = rdma

        out_ref[pl.ds(my_z * m, m), :] = x_ref[:, pl.ds(my_z * n, n)]

        fwds = []
        for s in range(N_Z - 1):
            z_rdmas[(N_Z - 1) - s].wait_recv()
            p = lax.rem(my_z + s + 1, N_Z)
            rows = out_ref.at[pl.ds(p * m + r_me * qm, qm), :]
            for dev, ssem, rsem in (
                ((1 - my_x, my_y, my_z), x_send, x_recv),
                ((my_x, 1 - my_y, my_z), y_send, y_recv),
                ((1 - my_x, 1 - my_y, my_z), g_send, g_recv),
            ):
                fwd = pltpu.make_async_remote_copy(
                    src_ref=rows,
                    dst_ref=rows,
                    send_sem=ssem.at[s],
                    recv_sem=rsem.at[s],
                    device_id=dev,
                    device_id_type=pl.DeviceIdType.MESH,
                )
                fwd.start()
                fwds.append(fwd)

        for fwd in fwds:
            fwd.wait_recv()

        for d in range(1, N_Z):
            z_rdmas[d].wait_send()
        for fwd in fwds:
            fwd.wait_send()

    return pl.pallas_call(
        body,
        out_shape=jax.ShapeDtypeStruct((N_Z * m, n), x.dtype),
        in_specs=[pl.BlockSpec(memory_space=pltpu.VMEM)],
        out_specs=pl.BlockSpec(memory_space=pltpu.VMEM),
        scratch_shapes=[
            pltpu.SemaphoreType.DMA((N_Z - 1,)),
            pltpu.SemaphoreType.DMA((N_Z - 1,)),
            pltpu.SemaphoreType.DMA((N_Z - 1,)),
            pltpu.SemaphoreType.DMA((N_Z - 1,)),
            pltpu.SemaphoreType.DMA((N_Z - 1,)),
            pltpu.SemaphoreType.DMA((N_Z - 1,)),
            pltpu.SemaphoreType.DMA((N_Z - 1,)),
            pltpu.SemaphoreType.DMA((N_Z - 1,)),
        ],
        compiler_params=pltpu.CompilerParams(collective_id=0),
    )(x)


# device time: 52346 ns/iter; 1.5292x vs baseline; 1.1625x over previous
import jax
import jax.numpy as jnp
from jax import lax
from jax.experimental import pallas as pl
from jax.experimental.pallas import tpu as pltpu

N_Z = 4
N_REP = 4


def kernel(x):
    m, n_full = x.shape
    n = n_full // N_Z
    qm = m // N_REP
    hq = qm // 2

    def body(x_ref, out_ref, z_send, z_recv, xd_send, xd_recv,
             yd_send, yd_recv, xr_send, xr_recv, yr_send, yr_recv):
        my_x = lax.axis_index("x")
        my_y = lax.axis_index("y")
        my_z = lax.axis_index("z")
        r_me = 2 * my_x + my_y
        r_xn = 2 * (1 - my_x) + my_y
        r_yn = 2 * my_x + (1 - my_y)
        x_nbr = (1 - my_x, my_y, my_z)
        y_nbr = (my_x, 1 - my_y, my_z)

        barrier_sem = pltpu.get_barrier_semaphore()
        for d in range(1, N_Z):
            q = lax.rem(my_z + d, N_Z)
            pl.semaphore_signal(
                barrier_sem, inc=1,
                device_id=(my_x, my_y, q),
                device_id_type=pl.DeviceIdType.MESH,
            )
        for dev in (x_nbr, y_nbr):
            pl.semaphore_signal(
                barrier_sem, inc=1,
                device_id=dev,
                device_id_type=pl.DeviceIdType.MESH,
            )
        pl.semaphore_wait(barrier_sem, 5)

        z_rdmas = {}
        for d in range(1, N_Z):
            q = lax.rem(my_z + d, N_Z)
            rdma = pltpu.make_async_remote_copy(
                src_ref=x_ref.at[pl.ds(r_me * qm, qm), pl.ds(q * n, n)],
                dst_ref=out_ref.at[pl.ds(my_z * m + r_me * qm, qm), :],
                send_sem=z_send.at[d - 1],
                recv_sem=z_recv.at[(N_Z - 1) - d],
                device_id=(my_x, my_y, q),
                device_id_type=pl.DeviceIdType.MESH,
            )
            rdma.start()
            z_rdmas[d] = rdma

        out_ref[pl.ds(my_z * m, m), :] = x_ref[:, pl.ds(my_z * n, n)]

        xds, yds = [], []
        for s in range(N_Z - 1):
            z_rdmas[(N_Z - 1) - s].wait_recv()
            p = lax.rem(my_z + s + 1, N_Z)
            rows = out_ref.at[pl.ds(p * m + r_me * qm, qm), :]
            for dev, ssem, rsem, acc in (
                (x_nbr, xd_send, xd_recv, xds),
                (y_nbr, yd_send, yd_recv, yds),
            ):
                fwd = pltpu.make_async_remote_copy(
                    src_ref=rows,
                    dst_ref=rows,
                    send_sem=ssem.at[s],
                    recv_sem=rsem.at[s],
                    device_id=dev,
                    device_id_type=pl.DeviceIdType.MESH,
                )
                fwd.start()
                acc.append(fwd)

        xrs, yrs = [], []
        for s in range(N_Z - 1):
            p = lax.rem(my_z + s + 1, N_Z)
            xds[s].wait_recv()
            rows = out_ref.at[pl.ds(p * m + r_xn * qm, hq), :]
            yr = pltpu.make_async_remote_copy(
                src_ref=rows,
                dst_ref=rows,
                send_sem=yr_send.at[s],
                recv_sem=yr_recv.at[s],
                device_id=y_nbr,
                device_id_type=pl.DeviceIdType.MESH,
            )
            yr.start()
            yrs.append(yr)

            yds[s].wait_recv()
            rows = out_ref.at[pl.ds(p * m + r_yn * qm + hq, hq), :]
            xr = pltpu.make_async_remote_copy(
                src_ref=rows,
                dst_ref=rows,
                send_sem=xr_send.at[s],
                recv_sem=xr_recv.at[s],
                device_id=x_nbr,
                device_id_type=pl.DeviceIdType.MESH,
            )
            xr.start()
            xrs.append(xr)

        for s in range(N_Z - 1):
            xrs[s].wait_recv()
            yrs[s].wait_recv()

        for d in range(1, N_Z):
            z_rdmas[d].wait_send()
        for fwd in xds + yds + xrs + yrs:
            fwd.wait_send()

    return pl.pallas_call(
        body,
        out_shape=jax.ShapeDtypeStruct((N_Z * m, n), x.dtype),
        in_specs=[pl.BlockSpec(memory_space=pltpu.VMEM)],
        out_specs=pl.BlockSpec(memory_space=pltpu.VMEM),
        scratch_shapes=[
            pltpu.SemaphoreType.DMA((N_Z - 1,)),
            pltpu.SemaphoreType.DMA((N_Z - 1,)),
            pltpu.SemaphoreType.DMA((N_Z - 1,)),
            pltpu.SemaphoreType.DMA((N_Z - 1,)),
            pltpu.SemaphoreType.DMA((N_Z - 1,)),
            pltpu.SemaphoreType.DMA((N_Z - 1,)),
            pltpu.SemaphoreType.DMA((N_Z - 1,)),
            pltpu.SemaphoreType.DMA((N_Z - 1,)),
            pltpu.SemaphoreType.DMA((N_Z - 1,)),
            pltpu.SemaphoreType.DMA((N_Z - 1,)),
        ],
        compiler_params=pltpu.CompilerParams(collective_id=0),
    )(x)
